# baseline (speedup 1.0000x reference)
import jax
import jax.numpy as jnp
from jax.experimental import pallas as pl
from jax.experimental.pallas import tpu as pltpu


def _make_kernel(BB, C, NC, C_out, HW):
    def _cond_conv_kernel(t_ref,
                          x_ref,
                          w_ref,
                          lab_ref,
                          btp_ref,
                          o_ref):
        g = pl.program_id(0)
        wx = w_ref[:, :C].astype(jnp.bfloat16)
        wctx = w_ref[:, C:]
        for j in range(BB):
            b = g * BB + j
            lab = lab_ref[pl.ds(b, 1), :]
            cond = jnp.sum(wctx * lab, axis=-1, keepdims=True)
            t_b = t_ref[b].astype(jnp.float32)
            cond = cond + btp_ref[:, 0:1] + t_b * btp_ref[:, 1:2]
            out = jnp.dot(wx, x_ref[j],
                          preferred_element_type=jnp.float32)
            o_ref[j] = (out + cond).astype(o_ref.dtype)
    return _cond_conv_kernel


def kernel(x, t, class_labels, w, bias, tproj):
    B, C, H, W = x.shape
    NC = class_labels.shape[1]
    C_out = w.shape[0]
    HW = H * W
    BB = 4

    x3d = x.astype(jnp.bfloat16).reshape(B, C, HW)
    btp = jnp.concatenate([bias, tproj], axis=1)
    grid = (B // BB,)

    out3d = pl.pallas_call(
        _make_kernel(BB, C, NC, C_out, HW),
        out_shape=jax.ShapeDtypeStruct((B, C_out, HW), jnp.bfloat16),
        grid=grid,
        in_specs=[
            pl.BlockSpec(memory_space=pltpu.SMEM),
            pl.BlockSpec((BB, C, HW), lambda g: (g, 0, 0)),
            pl.BlockSpec((C_out, C + NC), lambda g: (0, 0)),
            pl.BlockSpec((B, NC), lambda g: (0, 0)),
            pl.BlockSpec((C_out, 2), lambda g: (0, 0)),
        ],
        out_specs=pl.BlockSpec((BB, C_out, HW), lambda g: (g, 0, 0)),
        compiler_params=pltpu.CompilerParams(
            dimension_semantics=("parallel",)),
    )(t, x3d, w, class_labels, btp)

    return out3d.astype(x.dtype).reshape(B, C_out, H, W)


# speedup vs baseline: 1.0432x; 1.0432x over previous
import jax
import jax.numpy as jnp
from jax.experimental import pallas as pl
from jax.experimental.pallas import tpu as pltpu


def _make_kernel(BB, C, NC, C_out, HW):
    def _cond_conv_kernel(t_ref,
                          x_ref,
                          w_ref,
                          lab_ref,
                          btp_ref,
                          o_ref):
        g = pl.program_id(0)
        wx = w_ref[:, :C].astype(jnp.bfloat16)
        wctx = w_ref[:, C:]
        for j in range(BB):
            b = g * BB + j
            lab = lab_ref[pl.ds(b, 1), :]
            cond = jnp.sum(wctx * lab, axis=-1, keepdims=True)
            t_b = t_ref[b].astype(jnp.float32)
            cond = cond + btp_ref[:, 0:1] + t_b * btp_ref[:, 1:2]
            out = jnp.dot(wx, x_ref[j].astype(jnp.bfloat16),
                          preferred_element_type=jnp.float32)
            o_ref[j] = (out + cond).astype(o_ref.dtype)
    return _cond_conv_kernel


def kernel(x, t, class_labels, w, bias, tproj):
    B, C, H, W = x.shape
    NC = class_labels.shape[1]
    C_out = w.shape[0]
    HW = H * W
    BB = 4
    T_HW = HW // 2

    x3d = x.reshape(B, C, HW)
    btp = jnp.concatenate([bias, tproj], axis=1)
    grid = (B // BB, HW // T_HW)

    out3d = pl.pallas_call(
        _make_kernel(BB, C, NC, C_out, HW),
        out_shape=jax.ShapeDtypeStruct((B, C_out, HW), jnp.bfloat16),
        grid=grid,
        in_specs=[
            pl.BlockSpec(memory_space=pltpu.SMEM),
            pl.BlockSpec((BB, C, T_HW), lambda g, h: (g, 0, h)),
            pl.BlockSpec((C_out, C + NC), lambda g, h: (0, 0)),
            pl.BlockSpec((B, NC), lambda g, h: (0, 0)),
            pl.BlockSpec((C_out, 2), lambda g, h: (0, 0)),
        ],
        out_specs=pl.BlockSpec((BB, C_out, T_HW), lambda g, h: (g, 0, h)),
        compiler_params=pltpu.CompilerParams(
            dimension_semantics=("parallel", "parallel")),
    )(t, x3d, w, class_labels, btp)

    return out3d.astype(x.dtype).reshape(B, C_out, H, W)


# speedup vs baseline: 1.0747x; 1.0302x over previous
import jax
import jax.numpy as jnp
from jax.experimental import pallas as pl
from jax.experimental.pallas import tpu as pltpu


def _make_kernel(BB, C, NC, C_out, HW):
    def _cond_conv_kernel(t_ref,
                          x_ref,
                          w_ref,
                          lab_ref,
                          btp_ref,
                          o_ref):
        g = pl.program_id(0)
        wx = w_ref[:, :C].astype(jnp.bfloat16)
        wctx = w_ref[:, C:]
        for j in range(BB):
            b = g * BB + j
            lab = lab_ref[pl.ds(b, 1), :]
            cond = jnp.sum(wctx * lab, axis=-1, keepdims=True)
            t_b = t_ref[b].astype(jnp.float32)
            cond = cond + btp_ref[:, 0:1] + t_b * btp_ref[:, 1:2]
            out = jnp.dot(wx, x_ref[j].astype(jnp.bfloat16),
                          preferred_element_type=jnp.float32)
            o_ref[j] = (out + cond).astype(o_ref.dtype)
    return _cond_conv_kernel


def kernel(x, t, class_labels, w, bias, tproj):
    B, C, H, W = x.shape
    NC = class_labels.shape[1]
    C_out = w.shape[0]
    HW = H * W
    BB = 4
    T_HW = HW

    x3d = x.reshape(B, C, HW)
    btp = jnp.concatenate([bias, tproj], axis=1)
    grid = (B // BB, HW // T_HW)

    out3d = pl.pallas_call(
        _make_kernel(BB, C, NC, C_out, HW),
        out_shape=jax.ShapeDtypeStruct((B, C_out, HW), jnp.bfloat16),
        grid=grid,
        in_specs=[
            pl.BlockSpec(memory_space=pltpu.SMEM),
            pl.BlockSpec((BB, C, T_HW), lambda g, h: (g, 0, h)),
            pl.BlockSpec((C_out, C + NC), lambda g, h: (0, 0)),
            pl.BlockSpec((B, NC), lambda g, h: (0, 0)),
            pl.BlockSpec((C_out, 2), lambda g, h: (0, 0)),
        ],
        out_specs=pl.BlockSpec((BB, C_out, T_HW), lambda g, h: (g, 0, h)),
        compiler_params=pltpu.CompilerParams(
            dimension_semantics=("parallel", "parallel")),
    )(t, x3d, w, class_labels, btp)

    return out3d.astype(x.dtype).reshape(B, C_out, H, W)
